# baseline (speedup 1.0000x reference)
import functools

import jax
import jax.numpy as jnp
from jax import lax
from jax.experimental import pallas as pl
from jax.experimental.pallas import tpu as pltpu
from jax.experimental.pallas import tpu_sc as plsc

R_MAX = 5.0
AVG_NEIGH = 16.0
NB = 8
NSH = 16

NC = 2
NS = 16
NW = NC * NS

_SQRT2R = 0.6324555320336759
_PI = 3.14159265358979323846
_HALF_PI = 1.57079632679489661923


def _silu(x):
    return x * jax.nn.sigmoid(x)


def _sincos_pi(t):
    y = t * t
    st = t * (1.0 + y * (-1.0 / 6.0 + y * (1.0 / 120.0 + y * (-1.0 / 5040.0 + y * (1.0 / 362880.0)))))
    ct = 1.0 + y * (-0.5 + y * (1.0 / 24.0 + y * (-1.0 / 720.0 + y * (1.0 / 40320.0 - y * (1.0 / 3628800.0)))))
    return ct, -st


def _geom_body(px_h, py_h, pz_h, sx_h, sy_h, sz_h, snd_h, rcv_h, ef_h, shc_h,
               px_v, py_v, pz_v, shx, shy, shz, sidx, ridx, ef_b, shc_b):
    cid = lax.axis_index("c")
    sid = lax.axis_index("s")
    wid = sid * NC + cid
    E = snd_h.shape[0]
    epw = E // NW
    CG = 2000
    nchunk = epw // CG

    pltpu.sync_copy(px_h, px_v)
    pltpu.sync_copy(py_h, py_v)
    pltpu.sync_copy(pz_h, pz_v)

    def chunk(c, _):
        base = wid * epw + c * CG
        pltpu.sync_copy(snd_h.at[pl.ds(base, CG)], sidx)
        pltpu.sync_copy(rcv_h.at[pl.ds(base, CG)], ridx)
        pltpu.sync_copy(sx_h.at[pl.ds(base, CG)], shx)
        pltpu.sync_copy(sy_h.at[pl.ds(base, CG)], shy)
        pltpu.sync_copy(sz_h.at[pl.ds(base, CG)], shz)

        def grp(k, _):
            si = sidx[pl.ds(k * 16, 16)]
            ri = ridx[pl.ds(k * 16, 16)]
            sx = plsc.load_gather(px_v, [si])
            sy = plsc.load_gather(py_v, [si])
            sz = plsc.load_gather(pz_v, [si])
            rx = plsc.load_gather(px_v, [ri])
            ry = plsc.load_gather(py_v, [ri])
            rz = plsc.load_gather(pz_v, [ri])
            dx = rx - sx + shx[pl.ds(k * 16, 16)]
            dy = ry - sy + shy[pl.ds(k * 16, 16)]
            dz = rz - sz + shz[pl.ds(k * 16, 16)]
            r2 = dx * dx + dy * dy + dz * dz + 1e-12
            i32 = plsc.bitcast(r2, jnp.int32)
            inv = plsc.bitcast(0x5F3759DF - lax.shift_right_arithmetic(i32, 1),
                               jnp.float32)
            half = 0.5 * r2
            for _ in range(3):
                inv = inv * (1.5 - half * inv * inv)
            r = r2 * inv
            ux = dx * inv
            uy = dy * inv
            uz = dz * inv
            ur = r * (1.0 / R_MAX)
            ur2 = ur * ur
            ur4 = ur2 * ur2
            ur5 = ur4 * ur
            cutp = 1.0 + ur5 * (-21.0 + ur * (35.0 - 15.0 * ur))
            cut = jnp.where(ur < 1.0, cutp, 0.0)
            s1, c1 = _sincos_pi(_PI * ur - _HALF_PI)
            two_c1 = 2.0 * c1
            scale = _SQRT2R * inv * cut
            idx_base = (k * 16 + lax.iota(jnp.int32, 16)) * NB
            sm2 = s1
            sm1 = two_c1 * s1
            plsc.store_scatter(ef_b, [idx_base], scale * sm2)
            plsc.store_scatter(ef_b, [idx_base + 1], scale * sm1)
            for n in range(2, NB):
                sn = two_c1 * sm1 - sm2
                sm2 = sm1
                sm1 = sn
                plsc.store_scatter(ef_b, [idx_base + n], scale * sn)
            xx = ux * ux
            yy = uy * uy
            zz = uz * uz
            xy = ux * uy
            yz = uy * uz
            xz = ux * uz
            sh = [
                jnp.full((16,), 1.0, jnp.float32),
                1.7320508 * ux, 1.7320508 * uy, 1.7320508 * uz,
                3.8729833 * xy, 3.8729833 * yz, 1.1180340 * (3.0 * zz - 1.0),
                3.8729833 * xz, 1.9364917 * (xx - yy),
                2.0916500 * uy * (3.0 * xx - yy), 10.246951 * xy * uz,
                1.6201852 * uy * (5.0 * zz - 1.0), 1.3228757 * uz * (5.0 * zz - 3.0),
                1.6201852 * ux * (5.0 * zz - 1.0), 5.1234753 * uz * (xx - yy),
                2.0916500 * ux * (xx - 3.0 * yy),
            ]
            sidx_base = (k * 16 + lax.iota(jnp.int32, 16)) * NSH
            for kk in range(NSH):
                plsc.store_scatter(shc_b, [sidx_base + kk], sh[kk] * cut)
            return 0

        lax.fori_loop(0, CG // 16, grp, 0)
        pltpu.sync_copy(ef_b, ef_h.at[pl.ds(base * NB, CG * NB)])
        pltpu.sync_copy(shc_b, shc_h.at[pl.ds(base * NSH, CG * NSH)])
        return 0

    lax.fori_loop(0, nchunk, chunk, 0)


def _run_geom(px, py, pz, sx, sy, sz, snd, rcv):
    N = px.shape[0]
    E = snd.shape[0]
    CG = 2000
    mesh = plsc.VectorSubcoreMesh(core_axis_name="c", subcore_axis_name="s")
    f = pl.kernel(
        _geom_body,
        out_type=[
            jax.ShapeDtypeStruct((E * NB,), jnp.float32),
            jax.ShapeDtypeStruct((E * NSH,), jnp.float32),
        ],
        mesh=mesh,
        compiler_params=pltpu.CompilerParams(needs_layout_passes=False),
        scratch_types=[
            pltpu.VMEM((N,), jnp.float32),
            pltpu.VMEM((N,), jnp.float32),
            pltpu.VMEM((N,), jnp.float32),
            pltpu.VMEM((CG,), jnp.float32),
            pltpu.VMEM((CG,), jnp.float32),
            pltpu.VMEM((CG,), jnp.float32),
            pltpu.VMEM((CG,), jnp.int32),
            pltpu.VMEM((CG,), jnp.int32),
            pltpu.VMEM((CG * NB,), jnp.float32),
            pltpu.VMEM((CG * NSH,), jnp.float32),
        ],
    )
    ef, shc = f(px, py, pz, sx, sy, sz, snd, rcv)
    return ef.reshape(E, NB), shc.reshape(E, NSH)


def _msg_body(nf_h, w_h, snd_h, rcv_h, z_h, out_h,
              agg, sidx, ridx, r0, r1, wb0, wb1,
              rb0, rb1,
              g0, g1, q0, q1, s0, s1):
    cid = lax.axis_index("c")
    sid = lax.axis_index("s")
    wid = sid * NC + cid
    N = nf_h.shape[0]
    E = snd_h.shape[0]
    epw = E // NW
    C = rb0.shape[0]
    nfull = epw // C
    FH = nf_h.shape[1]
    rows = (r0, r1)
    wbuf = (wb0, wb1)
    rbi = (rb0, rb1)
    gsem = (g0, g1)
    wsem = (q0, q1)
    ssem = (s0, s1)
    npt = (N // NS) // 8 * 8
    rem = N - npt * NS

    pltpu.sync_copy(z_h.at[pl.ds(0, npt), :], agg.at[pl.ds(sid * npt, npt), :])

    @pl.when(sid == NS - 1)
    def _():
        pltpu.sync_copy(z_h.at[pl.ds(0, rem), :],
                        agg.at[pl.ds(npt * NS, rem), :])

    pltpu.sync_copy(snd_h.at[pl.ds(wid * epw, epw)], sidx)
    pltpu.sync_copy(rcv_h.at[pl.ds(wid * epw, epw)], ridx)

    def issue_gw(i, b):
        pltpu.async_copy(nf_h.at[sidx.at[pl.ds(i * C, C)]], rows[b], gsem[b])
        pltpu.async_copy(w_h.at[pl.ds(wid * epw + i * C, C), :],
                         wbuf[b], wsem[b])

    def wait_gw(b):
        pltpu.make_async_copy(nf_h.at[rbi[b]], rows[b], gsem[b]).wait()
        pltpu.make_async_copy(w_h.at[pl.ds(0, C), :], wbuf[b], wsem[b]).wait()

    def issue_scat(b):
        pltpu.async_copy(rows[b], agg.at[rbi[b]], ssem[b], add=True)

    def wait_scat(b):
        pltpu.make_async_copy(rows[b], agg.at[rbi[b]], ssem[b]).wait()

    def step(i, b, first):
        wait_gw(b)
        for off in (0, 16, C - 16):
            rbi[b][pl.ds(off, 16)] = ridx[pl.ds(i * C + off, 16)]
        b2 = 1 - b

        @pl.when(i + 1 < nfull)
        def _():
            if not first:
                wait_scat(b2)
            issue_gw(i + 1, b2)

        def mul(e, _):
            for j in range(FH // 16):
                sl = pl.ds(j * 16, 16)
                rows[b][e, sl] = rows[b][e, sl] * wbuf[b][e, sl]
            return 0
        lax.fori_loop(0, C, mul, 0)
        issue_scat(b)

    plsc.subcore_barrier()
    issue_gw(0, 0)
    step(0, 0, True)
    step(1, 1, False)

    def outer(k, _):
        step(2 * k, 0, False)
        step(2 * k + 1, 1, False)
        return 0

    lax.fori_loop(1, nfull // 2, outer, 0)
    wait_scat(0)
    wait_scat(1)
    plsc.subcore_barrier()
    pltpu.sync_copy(agg.at[pl.ds(sid * npt, npt), :],
                    out_h.at[cid, pl.ds(sid * npt, npt), :])

    @pl.when(sid == NS - 1)
    def _():
        pltpu.sync_copy(agg.at[pl.ds(npt * NS, rem), :],
                        out_h.at[cid, pl.ds(npt * NS, rem), :])


def _run_msg(nf, w, snd, rcv, zeros_block):
    N, F = nf.shape
    E = snd.shape[0]
    epw = E // NW
    C = 40
    mesh = plsc.VectorSubcoreMesh(core_axis_name="c", subcore_axis_name="s")
    f = pl.kernel(
        _msg_body,
        out_type=jax.ShapeDtypeStruct((NC, N, F), jnp.float32),
        mesh=mesh,
        compiler_params=pltpu.CompilerParams(needs_layout_passes=False),
        scratch_types=(
            [pltpu.VMEM_SHARED((N, F), jnp.float32),
             pltpu.VMEM((epw,), jnp.int32),
             pltpu.VMEM((epw,), jnp.int32)]
            + [pltpu.VMEM((C, F), jnp.float32)] * 4
            + [pltpu.VMEM((C,), jnp.int32)] * 2
            + [pltpu.SemaphoreType.DMA] * 6
        ),
    )
    return f(nf, w, snd, rcv, zeros_block)


def _edge_body(ef_ref, shc_ref, wr1_ref, wr2_ref, wsh_ref, out_ref):
    ef = ef_ref[...]
    shc = shc_ref[...]
    h = _silu(jnp.dot(ef, wr1_ref[...], preferred_element_type=jnp.float32))
    tpw = jnp.dot(h, wr2_ref[...], preferred_element_type=jnp.float32)
    shp = jnp.dot(shc, wsh_ref[...], preferred_element_type=jnp.float32)
    out_ref[...] = tpw * shp


def _run_edge(ef, shc, W_r1_t, W_r2_t, W_sh_t):
    E = ef.shape[0]
    F = W_r2_t.shape[1]
    BE = 2560
    grid = (E // BE,)
    return pl.pallas_call(
        _edge_body,
        grid=grid,
        in_specs=[
            pl.BlockSpec((BE, NB), lambda i: (i, 0)),
            pl.BlockSpec((BE, NSH), lambda i: (i, 0)),
            pl.BlockSpec(W_r1_t.shape, lambda i: (0, 0)),
            pl.BlockSpec(W_r2_t.shape, lambda i: (0, 0)),
            pl.BlockSpec(W_sh_t.shape, lambda i: (0, 0)),
        ],
        out_specs=pl.BlockSpec((BE, F), lambda i: (i, 0)),
        out_shape=jax.ShapeDtypeStruct((E, F), jnp.float32),
    )(ef, shc, W_r1_t, W_r2_t, W_sh_t)


def _embed_body(na_ref, w_ref, out_ref):
    out_ref[...] = jnp.dot(na_ref[...], w_ref[...],
                           preferred_element_type=jnp.float32)


def _run_embed(node_attrs, W_embed):
    N, NEL = node_attrs.shape
    F = W_embed.shape[1]
    return pl.pallas_call(
        _embed_body,
        out_shape=jax.ShapeDtypeStruct((N, F), jnp.float32),
    )(node_attrs, W_embed)


def _node_body(final, agg2_ref, na_ref, nf_ref, pos_ref, p_ref,
               welem_ref, wup_ref, wa_ref, wb_ref,
               nf_out_ref, scd_ref, scp_ref, spos_ref):
    agg = (agg2_ref[0] + agg2_ref[1]) * (1.0 / AVG_NEIGH)
    elem = jnp.dot(na_ref[...], welem_ref[...], preferred_element_type=jnp.float32)
    aggs = agg * elem
    upd = _silu(jnp.dot(aggs, wup_ref[...], preferred_element_type=jnp.float32))
    nf2 = nf_ref[...] + upd
    nf_out_ref[...] = nf2
    if final:
        hm = _silu(jnp.dot(nf2, wa_ref[...], preferred_element_type=jnp.float32))
        out = jnp.dot(hm, wb_ref[...], preferred_element_type=jnp.float32)
    else:
        out = jnp.dot(nf2, wa_ref[...], preferred_element_type=jnp.float32)
    c = out[:, 0:1]
    pos = pos_ref[...]
    p = p_ref[...]
    scd_ref[...] = jnp.dot(p, out, preferred_element_type=jnp.float32)
    scp_ref[...] = jnp.dot(p, c * pos, preferred_element_type=jnp.float32)
    spos_ref[...] = jnp.dot(p, pos, preferred_element_type=jnp.float32)


def _run_node(final, agg2, node_attrs, nf, positions, pmat,
              W_elem_t, W_up_t, wa, wb):
    N, F = nf.shape
    G = pmat.shape[0]
    return pl.pallas_call(
        functools.partial(_node_body, final),
        out_shape=[
            jax.ShapeDtypeStruct((N, F), jnp.float32),
            jax.ShapeDtypeStruct((G, 4), jnp.float32),
            jax.ShapeDtypeStruct((G, 3), jnp.float32),
            jax.ShapeDtypeStruct((G, 3), jnp.float32),
        ],
    )(agg2, node_attrs, nf, positions, pmat, W_elem_t, W_up_t, wa, wb)


def kernel(positions, node_attrs, shifts, total_charge, W_embed, W_r1, W_r2,
           W_sh, W_elem, W_up, W_read, W_mlp1, W_mlp2, edge_index, batch, ptr):
    N = positions.shape[0]
    E = edge_index.shape[1]
    F = W_embed.shape[1]
    snd = edge_index[0]
    rcv = edge_index[1]
    px = positions[:, 0]
    py = positions[:, 1]
    pz = positions[:, 2]

    ef, shc = _run_geom(px, py, pz, shifts[:, 0], shifts[:, 1], shifts[:, 2],
                        snd, rcv)
    w0 = _run_edge(ef, shc, W_r1[0], W_r2[0], W_sh[0])
    w1 = _run_edge(ef, shc, W_r1[1], W_r2[1], W_sh[1])
    nf0 = _run_embed(node_attrs, W_embed)

    zeros_block = jnp.zeros((N // NS, F), jnp.float32)
    W_read4 = W_read[:, jnp.array([0, 2, 3, 4])]
    W_mlp2_4 = W_mlp2[:, jnp.array([0, 2, 3, 4])]
    G = total_charge.shape[0]
    pmat = (batch[None, :] == jnp.arange(G, dtype=batch.dtype)[:, None]
            ).astype(jnp.float32)

    agg2_0 = _run_msg(nf0, w0, snd, rcv, zeros_block)
    nf1, scd0, scp0, spos = _run_node(
        False, agg2_0, node_attrs, nf0, positions, pmat,
        W_elem[0], W_up[0], W_read4, W_read4)
    agg2_1 = _run_msg(nf1, w1, snd, rcv, zeros_block)
    _, scd1, scp1, _ = _run_node(
        True, agg2_1, node_attrs, nf1, positions, pmat,
        W_elem[1], W_up[1], W_mlp1, W_mlp2_4)

    sum_c = scd0[:, 0] + scd1[:, 0]
    sum_d = scd0[:, 1:4] + scd1[:, 1:4]
    sum_cpos = scp0 + scp1
    num = (ptr[1:] - ptr[:-1]).astype(jnp.float32)
    excess = (sum_c - total_charge) / num
    return sum_d + sum_cpos - excess[:, None] * spos


# speedup vs baseline: 1.0258x; 1.0258x over previous
import functools

import jax
import jax.numpy as jnp
from jax import lax
from jax.experimental import pallas as pl
from jax.experimental.pallas import tpu as pltpu
from jax.experimental.pallas import tpu_sc as plsc

R_MAX = 5.0
AVG_NEIGH = 16.0
NB = 8
NSH = 16

NC = 2
NS = 16
NW = NC * NS

_SQRT2R = 0.6324555320336759
_PI = 3.14159265358979323846
_HALF_PI = 1.57079632679489661923


def _silu(x):
    return x * jax.nn.sigmoid(x)


def _sincos_pi(t):
    y = t * t
    st = t * (1.0 + y * (-1.0 / 6.0 + y * (1.0 / 120.0 + y * (-1.0 / 5040.0 + y * (1.0 / 362880.0)))))
    ct = 1.0 + y * (-0.5 + y * (1.0 / 24.0 + y * (-1.0 / 720.0 + y * (1.0 / 40320.0 - y * (1.0 / 3628800.0)))))
    return ct, -st


def _geom_body(px_h, py_h, pz_h, sx_h, sy_h, sz_h, snd_h, rcv_h, ef_h, shc_h,
               px_v, py_v, pz_v, shx, shy, shz, sidx, ridx, ef_b, shc_b):
    cid = lax.axis_index("c")
    sid = lax.axis_index("s")
    wid = sid * NC + cid
    E = snd_h.shape[0]
    epw = E // NW
    CG = 2000
    nchunk = epw // CG

    pltpu.sync_copy(px_h, px_v)
    pltpu.sync_copy(py_h, py_v)
    pltpu.sync_copy(pz_h, pz_v)

    def chunk(c, _):
        base = wid * epw + c * CG
        pltpu.sync_copy(snd_h.at[pl.ds(base, CG)], sidx)
        pltpu.sync_copy(rcv_h.at[pl.ds(base, CG)], ridx)
        pltpu.sync_copy(sx_h.at[pl.ds(base, CG)], shx)
        pltpu.sync_copy(sy_h.at[pl.ds(base, CG)], shy)
        pltpu.sync_copy(sz_h.at[pl.ds(base, CG)], shz)

        def grp(k, _):
            si = sidx[pl.ds(k * 16, 16)]
            ri = ridx[pl.ds(k * 16, 16)]
            sx = plsc.load_gather(px_v, [si])
            sy = plsc.load_gather(py_v, [si])
            sz = plsc.load_gather(pz_v, [si])
            rx = plsc.load_gather(px_v, [ri])
            ry = plsc.load_gather(py_v, [ri])
            rz = plsc.load_gather(pz_v, [ri])
            dx = rx - sx + shx[pl.ds(k * 16, 16)]
            dy = ry - sy + shy[pl.ds(k * 16, 16)]
            dz = rz - sz + shz[pl.ds(k * 16, 16)]
            r2 = dx * dx + dy * dy + dz * dz + 1e-12
            i32 = plsc.bitcast(r2, jnp.int32)
            inv = plsc.bitcast(0x5F3759DF - lax.shift_right_arithmetic(i32, 1),
                               jnp.float32)
            half = 0.5 * r2
            for _ in range(3):
                inv = inv * (1.5 - half * inv * inv)
            r = r2 * inv
            ux = dx * inv
            uy = dy * inv
            uz = dz * inv
            ur = r * (1.0 / R_MAX)
            ur2 = ur * ur
            ur4 = ur2 * ur2
            ur5 = ur4 * ur
            cutp = 1.0 + ur5 * (-21.0 + ur * (35.0 - 15.0 * ur))
            cut = jnp.where(ur < 1.0, cutp, 0.0)
            s1, c1 = _sincos_pi(_PI * ur - _HALF_PI)
            two_c1 = 2.0 * c1
            scale = _SQRT2R * inv * cut
            idx_base = (k * 16 + lax.iota(jnp.int32, 16)) * NB
            sm2 = s1
            sm1 = two_c1 * s1
            plsc.store_scatter(ef_b, [idx_base], scale * sm2)
            plsc.store_scatter(ef_b, [idx_base + 1], scale * sm1)
            for n in range(2, NB):
                sn = two_c1 * sm1 - sm2
                sm2 = sm1
                sm1 = sn
                plsc.store_scatter(ef_b, [idx_base + n], scale * sn)
            xx = ux * ux
            yy = uy * uy
            zz = uz * uz
            xy = ux * uy
            yz = uy * uz
            xz = ux * uz
            sh = [
                jnp.full((16,), 1.0, jnp.float32),
                1.7320508 * ux, 1.7320508 * uy, 1.7320508 * uz,
                3.8729833 * xy, 3.8729833 * yz, 1.1180340 * (3.0 * zz - 1.0),
                3.8729833 * xz, 1.9364917 * (xx - yy),
                2.0916500 * uy * (3.0 * xx - yy), 10.246951 * xy * uz,
                1.6201852 * uy * (5.0 * zz - 1.0), 1.3228757 * uz * (5.0 * zz - 3.0),
                1.6201852 * ux * (5.0 * zz - 1.0), 5.1234753 * uz * (xx - yy),
                2.0916500 * ux * (xx - 3.0 * yy),
            ]
            sidx_base = (k * 16 + lax.iota(jnp.int32, 16)) * NSH
            for kk in range(NSH):
                plsc.store_scatter(shc_b, [sidx_base + kk], sh[kk] * cut)
            return 0

        lax.fori_loop(0, CG // 16, grp, 0)
        pltpu.sync_copy(ef_b, ef_h.at[pl.ds(base * NB, CG * NB)])
        pltpu.sync_copy(shc_b, shc_h.at[pl.ds(base * NSH, CG * NSH)])
        return 0

    lax.fori_loop(0, nchunk, chunk, 0)


def _run_geom(px, py, pz, sx, sy, sz, snd, rcv):
    N = px.shape[0]
    E = snd.shape[0]
    CG = 2000
    mesh = plsc.VectorSubcoreMesh(core_axis_name="c", subcore_axis_name="s")
    f = pl.kernel(
        _geom_body,
        out_type=[
            jax.ShapeDtypeStruct((E * NB,), jnp.float32),
            jax.ShapeDtypeStruct((E * NSH,), jnp.float32),
        ],
        mesh=mesh,
        compiler_params=pltpu.CompilerParams(needs_layout_passes=False),
        scratch_types=[
            pltpu.VMEM((N,), jnp.float32),
            pltpu.VMEM((N,), jnp.float32),
            pltpu.VMEM((N,), jnp.float32),
            pltpu.VMEM((CG,), jnp.float32),
            pltpu.VMEM((CG,), jnp.float32),
            pltpu.VMEM((CG,), jnp.float32),
            pltpu.VMEM((CG,), jnp.int32),
            pltpu.VMEM((CG,), jnp.int32),
            pltpu.VMEM((CG * NB,), jnp.float32),
            pltpu.VMEM((CG * NSH,), jnp.float32),
        ],
    )
    ef, shc = f(px, py, pz, sx, sy, sz, snd, rcv)
    return ef.reshape(E, NB), shc.reshape(E, NSH)


def _msg_body(nf_h, w_h, snd_h, rcv_h, z_h, out_h,
              agg, sidx, ridx, r0, r1, wb0, wb1,
              rb0, rb1,
              g0, g1, q0, q1, s0, s1):
    cid = lax.axis_index("c")
    sid = lax.axis_index("s")
    wid = sid * NC + cid
    N = nf_h.shape[0]
    E = snd_h.shape[0]
    epw = E // NW
    C = rb0.shape[0]
    nfull = epw // C
    FH = nf_h.shape[1]
    rows = (r0, r1)
    wbuf = (wb0, wb1)
    rbi = (rb0, rb1)
    gsem = (g0, g1)
    wsem = (q0, q1)
    ssem = (s0, s1)
    npt = (N // NS) // 8 * 8
    rem = N - npt * NS

    pltpu.sync_copy(z_h.at[pl.ds(0, npt), :], agg.at[pl.ds(sid * npt, npt), :])

    @pl.when(sid == NS - 1)
    def _():
        pltpu.sync_copy(z_h.at[pl.ds(0, rem), :],
                        agg.at[pl.ds(npt * NS, rem), :])

    pltpu.sync_copy(snd_h.at[pl.ds(wid * epw, epw)], sidx)
    pltpu.sync_copy(rcv_h.at[pl.ds(wid * epw, epw)], ridx)

    def issue_gw(i, b):
        pltpu.async_copy(nf_h.at[sidx.at[pl.ds(i * C, C)]], rows[b], gsem[b])
        pltpu.async_copy(w_h.at[pl.ds(wid * epw + i * C, C), :],
                         wbuf[b], wsem[b])

    def wait_gw(b):
        pltpu.make_async_copy(nf_h.at[rbi[b]], rows[b], gsem[b]).wait()
        pltpu.make_async_copy(w_h.at[pl.ds(0, C), :], wbuf[b], wsem[b]).wait()

    def issue_scat(b):
        pltpu.async_copy(rows[b], agg.at[rbi[b]], ssem[b], add=True)

    def wait_scat(b):
        pltpu.make_async_copy(rows[b], agg.at[rbi[b]], ssem[b]).wait()

    def step(i, b, first):
        wait_gw(b)
        for off in (0, 16, C - 16):
            rbi[b][pl.ds(off, 16)] = ridx[pl.ds(i * C + off, 16)]
        b2 = 1 - b

        @pl.when(i + 1 < nfull)
        def _():
            if not first:
                wait_scat(b2)
            issue_gw(i + 1, b2)

        def mul(e, _):
            for j in range(FH // 16):
                sl = pl.ds(j * 16, 16)
                rows[b][e, sl] = rows[b][e, sl] * wbuf[b][e, sl]
            return 0
        lax.fori_loop(0, C, mul, 0)
        issue_scat(b)

    plsc.subcore_barrier()
    issue_gw(0, 0)
    step(0, 0, True)
    step(1, 1, False)

    def outer(k, _):
        step(2 * k, 0, False)
        step(2 * k + 1, 1, False)
        return 0

    lax.fori_loop(1, nfull // 2, outer, 0)
    wait_scat(0)
    wait_scat(1)
    plsc.subcore_barrier()
    pltpu.sync_copy(agg.at[pl.ds(sid * npt, npt), :],
                    out_h.at[cid, pl.ds(sid * npt, npt), :])

    @pl.when(sid == NS - 1)
    def _():
        pltpu.sync_copy(agg.at[pl.ds(npt * NS, rem), :],
                        out_h.at[cid, pl.ds(npt * NS, rem), :])


def _run_msg(nf, w, snd, rcv, zeros_block):
    N, F = nf.shape
    E = snd.shape[0]
    epw = E // NW
    C = 40
    mesh = plsc.VectorSubcoreMesh(core_axis_name="c", subcore_axis_name="s")
    f = pl.kernel(
        _msg_body,
        out_type=jax.ShapeDtypeStruct((NC, N, F), jnp.float32),
        mesh=mesh,
        compiler_params=pltpu.CompilerParams(needs_layout_passes=False),
        scratch_types=(
            [pltpu.VMEM_SHARED((N, F), jnp.float32),
             pltpu.VMEM((epw,), jnp.int32),
             pltpu.VMEM((epw,), jnp.int32)]
            + [pltpu.VMEM((C, F), jnp.float32)] * 4
            + [pltpu.VMEM((C,), jnp.int32)] * 2
            + [pltpu.SemaphoreType.DMA] * 6
        ),
    )
    return f(nf, w, snd, rcv, zeros_block)


def _edge_body(ef_ref, shc_ref, wr1_ref, wr2_ref, wsh_ref, w0_ref, w1_ref):
    ef = ef_ref[...]
    shc = shc_ref[...]
    for t, out in ((0, w0_ref), (1, w1_ref)):
        h = _silu(jnp.dot(ef, wr1_ref[t], preferred_element_type=jnp.float32))
        tpw = jnp.dot(h, wr2_ref[t], preferred_element_type=jnp.float32)
        shp = jnp.dot(shc, wsh_ref[t], preferred_element_type=jnp.float32)
        out[...] = tpw * shp


def _run_edge(ef, shc, W_r1, W_r2, W_sh):
    E = ef.shape[0]
    F = W_r2.shape[2]
    BE = 2560
    grid = (E // BE,)
    return pl.pallas_call(
        _edge_body,
        grid=grid,
        in_specs=[
            pl.BlockSpec((BE, NB), lambda i: (i, 0)),
            pl.BlockSpec((BE, NSH), lambda i: (i, 0)),
            pl.BlockSpec(W_r1.shape, lambda i: (0, 0, 0)),
            pl.BlockSpec(W_r2.shape, lambda i: (0, 0, 0)),
            pl.BlockSpec(W_sh.shape, lambda i: (0, 0, 0)),
        ],
        out_specs=[pl.BlockSpec((BE, F), lambda i: (i, 0))] * 2,
        out_shape=[jax.ShapeDtypeStruct((E, F), jnp.float32)] * 2,
    )(ef, shc, W_r1, W_r2, W_sh)


def _embed_body(na_ref, w_ref, out_ref):
    out_ref[...] = jnp.dot(na_ref[...], w_ref[...],
                           preferred_element_type=jnp.float32)


def _run_embed(node_attrs, W_embed):
    N, NEL = node_attrs.shape
    F = W_embed.shape[1]
    return pl.pallas_call(
        _embed_body,
        out_shape=jax.ShapeDtypeStruct((N, F), jnp.float32),
    )(node_attrs, W_embed)


def _node_body(final, agg2_ref, na_ref, nf_ref, pos_ref, p_ref,
               welem_ref, wup_ref, wa_ref, wb_ref,
               nf_out_ref, scd_ref, scp_ref, spos_ref):
    agg = (agg2_ref[0] + agg2_ref[1]) * (1.0 / AVG_NEIGH)
    elem = jnp.dot(na_ref[...], welem_ref[...], preferred_element_type=jnp.float32)
    aggs = agg * elem
    upd = _silu(jnp.dot(aggs, wup_ref[...], preferred_element_type=jnp.float32))
    nf2 = nf_ref[...] + upd
    nf_out_ref[...] = nf2
    if final:
        hm = _silu(jnp.dot(nf2, wa_ref[...], preferred_element_type=jnp.float32))
        out = jnp.dot(hm, wb_ref[...], preferred_element_type=jnp.float32)
    else:
        out = jnp.dot(nf2, wa_ref[...], preferred_element_type=jnp.float32)
    c = out[:, 0:1]
    pos = pos_ref[...]
    p = p_ref[...]
    scd_ref[...] = jnp.dot(p, out, preferred_element_type=jnp.float32)
    scp_ref[...] = jnp.dot(p, c * pos, preferred_element_type=jnp.float32)
    spos_ref[...] = jnp.dot(p, pos, preferred_element_type=jnp.float32)


def _run_node(final, agg2, node_attrs, nf, positions, pmat,
              W_elem_t, W_up_t, wa, wb):
    N, F = nf.shape
    G = pmat.shape[0]
    return pl.pallas_call(
        functools.partial(_node_body, final),
        out_shape=[
            jax.ShapeDtypeStruct((N, F), jnp.float32),
            jax.ShapeDtypeStruct((G, 4), jnp.float32),
            jax.ShapeDtypeStruct((G, 3), jnp.float32),
            jax.ShapeDtypeStruct((G, 3), jnp.float32),
        ],
    )(agg2, node_attrs, nf, positions, pmat, W_elem_t, W_up_t, wa, wb)


def kernel(positions, node_attrs, shifts, total_charge, W_embed, W_r1, W_r2,
           W_sh, W_elem, W_up, W_read, W_mlp1, W_mlp2, edge_index, batch, ptr):
    N = positions.shape[0]
    E = edge_index.shape[1]
    F = W_embed.shape[1]
    snd = edge_index[0]
    rcv = edge_index[1]
    px = positions[:, 0]
    py = positions[:, 1]
    pz = positions[:, 2]

    ef, shc = _run_geom(px, py, pz, shifts[:, 0], shifts[:, 1], shifts[:, 2],
                        snd, rcv)
    w0, w1 = _run_edge(ef, shc, W_r1, W_r2, W_sh)
    nf0 = _run_embed(node_attrs, W_embed)

    zeros_block = jnp.zeros((N // NS, F), jnp.float32)
    W_read4 = W_read[:, jnp.array([0, 2, 3, 4])]
    W_mlp2_4 = W_mlp2[:, jnp.array([0, 2, 3, 4])]
    G = total_charge.shape[0]
    pmat = (batch[None, :] == jnp.arange(G, dtype=batch.dtype)[:, None]
            ).astype(jnp.float32)

    agg2_0 = _run_msg(nf0, w0, snd, rcv, zeros_block)
    nf1, scd0, scp0, spos = _run_node(
        False, agg2_0, node_attrs, nf0, positions, pmat,
        W_elem[0], W_up[0], W_read4, W_read4)
    agg2_1 = _run_msg(nf1, w1, snd, rcv, zeros_block)
    _, scd1, scp1, _ = _run_node(
        True, agg2_1, node_attrs, nf1, positions, pmat,
        W_elem[1], W_up[1], W_mlp1, W_mlp2_4)

    sum_c = scd0[:, 0] + scd1[:, 0]
    sum_d = scd0[:, 1:4] + scd1[:, 1:4]
    sum_cpos = scp0 + scp1
    num = (ptr[1:] - ptr[:-1]).astype(jnp.float32)
    excess = (sum_c - total_charge) / num
    return sum_d + sum_cpos - excess[:, None] * spos
